# Initial kernel scaffold; baseline (speedup 1.0000x reference)
import jax
import jax.numpy as jnp
from jax.experimental import pallas as pl


def kernel(token_ids, table):
    raise NotImplementedError("write your pallas kernel here")


# speedup vs baseline: 8.7927x; 8.7927x over previous
import functools

import jax
import jax.numpy as jnp
from jax import lax
from jax.experimental import pallas as pl
from jax.experimental.pallas import tpu as pltpu
from jax.experimental.pallas import tpu_sc as plsc


def _make_gather(N, V, D, NC, NS, chunk):
    NW = NC * NS
    n_per_w = N // NW
    n_chunks = n_per_w // chunk
    mesh = plsc.VectorSubcoreMesh(core_axis_name="c", subcore_axis_name="s")

    @functools.partial(
        pl.kernel,
        mesh=mesh,
        out_type=jax.ShapeDtypeStruct((N, D), jnp.float32),
        scratch_types=[
            pltpu.VMEM((chunk,), jnp.int32),
            pltpu.VMEM((chunk, D), jnp.float32),
            pltpu.SemaphoreType.DMA,
        ],
    )
    def gather_kernel(idx_hbm, table_hbm, out_hbm, idx_v, rows_v, sem):
        wid = lax.axis_index("s") * NC + lax.axis_index("c")
        base = wid * n_per_w

        def body(c, carry):
            off = base + c * chunk
            pltpu.sync_copy(idx_hbm.at[pl.ds(off, chunk)], idx_v)
            pltpu.async_copy(table_hbm.at[idx_v], rows_v, sem).wait()
            pltpu.sync_copy(rows_v, out_hbm.at[pl.ds(off, chunk)])
            return carry

        lax.fori_loop(0, n_chunks, body, ())

    return gather_kernel


def kernel(token_ids, table):
    B, S = token_ids.shape
    V, D = table.shape
    N = B * S
    info = plsc.get_sparse_core_info()
    NC, NS = info.num_cores, info.num_subcores
    ids_flat = token_ids.reshape(N).astype(jnp.int32)
    out = _make_gather(N, V, D, NC, NS, chunk=800)(ids_flat, table)
    return out.reshape(B, S, D)


# speedup vs baseline: 8.9914x; 1.0226x over previous
import functools

import jax
import jax.numpy as jnp
from jax import lax
from jax.experimental import pallas as pl
from jax.experimental.pallas import tpu as pltpu
from jax.experimental.pallas import tpu_sc as plsc


def _make_gather(N, V, D, NC, NS, chunk):
    NW = NC * NS
    n_per_w = N // NW
    n_chunks = n_per_w // chunk
    n_groups = n_chunks // 2
    mesh = plsc.VectorSubcoreMesh(core_axis_name="c", subcore_axis_name="s")

    @functools.partial(
        pl.kernel,
        mesh=mesh,
        out_type=jax.ShapeDtypeStruct((N, D), jnp.float32),
        scratch_types=[
            pltpu.VMEM((chunk,), jnp.int32),
            pltpu.VMEM((chunk,), jnp.int32),
            pltpu.VMEM((chunk, D), jnp.float32),
            pltpu.VMEM((chunk, D), jnp.float32),
            pltpu.SemaphoreType.DMA,
            pltpu.SemaphoreType.DMA,
            pltpu.SemaphoreType.DMA,
        ],
    )
    def gather_kernel(idx_hbm, table_hbm, out_hbm, idx0, idx1, rows0, rows1,
                      sg, sw0, sw1):
        wid = lax.axis_index("s") * NC + lax.axis_index("c")
        base = wid * n_per_w
        idx_b = (idx0, idx1)
        rows_b = (rows0, rows1)
        sw = (sw0, sw1)

        def group(g, carry):
            for b in range(2):
                off = base + (g * 2 + b) * chunk

                @pl.when(g > 0)
                def _():
                    pltpu.make_async_copy(
                        rows_b[b], out_hbm.at[pl.ds(off, chunk)], sw[b]
                    ).wait()

                pltpu.sync_copy(idx_hbm.at[pl.ds(off, chunk)], idx_b[b])
                pltpu.async_copy(table_hbm.at[idx_b[b]], rows_b[b], sg).wait()
                pltpu.async_copy(
                    rows_b[b], out_hbm.at[pl.ds(off, chunk)], sw[b]
                )
            return carry

        lax.fori_loop(0, n_groups, group, ())

        for b in range(2):
            off = base + ((n_groups - 1) * 2 + b) * chunk
            pltpu.make_async_copy(
                rows_b[b], out_hbm.at[pl.ds(off, chunk)], sw[b]
            ).wait()

    return gather_kernel


def kernel(token_ids, table):
    B, S = token_ids.shape
    V, D = table.shape
    N = B * S
    info = plsc.get_sparse_core_info()
    NC, NS = info.num_cores, info.num_subcores
    ids_flat = token_ids.reshape(N).astype(jnp.int32)
    out = _make_gather(N, V, D, NC, NS, chunk=400)(ids_flat, table)
    return out.reshape(B, S, D)


# speedup vs baseline: 9.1782x; 1.0208x over previous
import functools

import jax
import jax.numpy as jnp
from jax import lax
from jax.experimental import pallas as pl
from jax.experimental.pallas import tpu as pltpu
from jax.experimental.pallas import tpu_sc as plsc


def _make_gather(N, V, D, NC, NS, chunk):
    NW = NC * NS
    n_per_w = N // NW
    n_chunks = n_per_w // chunk
    n_groups = n_chunks // 2
    mesh = plsc.VectorSubcoreMesh(core_axis_name="c", subcore_axis_name="s")

    @functools.partial(
        pl.kernel,
        mesh=mesh,
        out_type=jax.ShapeDtypeStruct((N, D), jnp.float32),
        scratch_types=[
            pltpu.VMEM((chunk,), jnp.int32),
            pltpu.VMEM((chunk,), jnp.int32),
            pltpu.VMEM((chunk, D), jnp.float32),
            pltpu.VMEM((chunk, D), jnp.float32),
            pltpu.SemaphoreType.DMA,
            pltpu.SemaphoreType.DMA,
            pltpu.SemaphoreType.DMA,
            pltpu.SemaphoreType.DMA,
            pltpu.SemaphoreType.DMA,
            pltpu.SemaphoreType.DMA,
        ],
    )
    def gather_kernel(idx_hbm, table_hbm, out_hbm, idx0, idx1, rows0, rows1,
                      sg0, sg1, sw0, sw1, si0, si1):
        wid = lax.axis_index("s") * NC + lax.axis_index("c")
        base = wid * n_per_w
        idx_b = (idx0, idx1)
        rows_b = (rows0, rows1)
        sg = (sg0, sg1)
        sw = (sw0, sw1)
        si = (si0, si1)

        def idx_src(c):
            return idx_hbm.at[pl.ds(base + c * chunk, chunk)]

        def out_dst(c):
            return out_hbm.at[pl.ds(base + c * chunk, chunk)]

        pltpu.async_copy(idx_src(0), idx0, si0)
        pltpu.async_copy(idx_src(1), idx1, si1)
        pltpu.make_async_copy(idx_src(0), idx0, si0).wait()
        pltpu.async_copy(table_hbm.at[idx0], rows0, sg0)

        def group(g, carry):
            for b in range(2):
                c = g * 2 + b
                nxt = 1 - b

                pltpu.make_async_copy(
                    table_hbm.at[idx_b[b]], rows_b[b], sg[b]
                ).wait()

                def start_next():
                    pltpu.make_async_copy(
                        idx_src(c + 1), idx_b[nxt], si[nxt]
                    ).wait()
                    pltpu.make_async_copy(
                        rows_b[nxt], out_dst(c - 1), sw[nxt]
                    ).wait()
                    pltpu.async_copy(
                        table_hbm.at[idx_b[nxt]], rows_b[nxt], sg[nxt]
                    )

                if b == 0:
                    @pl.when(g > 0)
                    def _():
                        pltpu.make_async_copy(
                            rows_b[nxt], out_dst(c - 1), sw[nxt]
                        ).wait()

                    pltpu.make_async_copy(
                        idx_src(c + 1), idx_b[nxt], si[nxt]
                    ).wait()
                    pltpu.async_copy(
                        table_hbm.at[idx_b[nxt]], rows_b[nxt], sg[nxt]
                    )
                else:
                    @pl.when(g < n_groups - 1)
                    def _():
                        start_next()

                pltpu.async_copy(rows_b[b], out_dst(c), sw[b])

                @pl.when(g < n_groups - 1)
                def _():
                    pltpu.async_copy(idx_src(c + 2), idx_b[b], si[b])

            return carry

        lax.fori_loop(0, n_groups, group, ())

        for b in range(2):
            off = base + ((n_groups - 1) * 2 + b) * chunk
            pltpu.make_async_copy(
                rows_b[b], out_hbm.at[pl.ds(off, chunk)], sw[b]
            ).wait()

    return gather_kernel


def kernel(token_ids, table):
    B, S = token_ids.shape
    V, D = table.shape
    N = B * S
    info = plsc.get_sparse_core_info()
    NC, NS = info.num_cores, info.num_subcores
    ids_flat = token_ids.reshape(N).astype(jnp.int32)
    out = _make_gather(N, V, D, NC, NS, chunk=400)(ids_flat, table)
    return out.reshape(B, S, D)


# speedup vs baseline: 9.1788x; 1.0001x over previous
import functools

import jax
import jax.numpy as jnp
from jax import lax
from jax.experimental import pallas as pl
from jax.experimental.pallas import tpu as pltpu
from jax.experimental.pallas import tpu_sc as plsc


def _make_gather(N, V, D, NC, NS, chunk):
    NW = NC * NS
    n_per_w = N // NW
    n_chunks = n_per_w // chunk
    n_groups = n_chunks // 2
    mesh = plsc.VectorSubcoreMesh(core_axis_name="c", subcore_axis_name="s")

    @functools.partial(
        pl.kernel,
        mesh=mesh,
        out_type=jax.ShapeDtypeStruct((N, D), jnp.float32),
        scratch_types=[
            pltpu.VMEM((chunk,), jnp.int32),
            pltpu.VMEM((chunk,), jnp.int32),
            pltpu.VMEM((chunk, D), jnp.float32),
            pltpu.VMEM((chunk, D), jnp.float32),
            pltpu.SemaphoreType.DMA,
            pltpu.SemaphoreType.DMA,
            pltpu.SemaphoreType.DMA,
            pltpu.SemaphoreType.DMA,
            pltpu.SemaphoreType.DMA,
            pltpu.SemaphoreType.DMA,
        ],
    )
    def gather_kernel(idx_hbm, table_hbm, out_hbm, idx0, idx1, rows0, rows1,
                      sg0, sg1, sw0, sw1, si0, si1):
        wid = lax.axis_index("s") * NC + lax.axis_index("c")
        base = wid * n_per_w
        idx_b = (idx0, idx1)
        rows_b = (rows0, rows1)
        sg = (sg0, sg1)
        sw = (sw0, sw1)
        si = (si0, si1)

        def idx_src(c):
            return idx_hbm.at[pl.ds(base + c * chunk, chunk)]

        def out_dst(c):
            return out_hbm.at[pl.ds(base + c * chunk, chunk)]

        pltpu.async_copy(idx_src(0), idx0, si0)
        pltpu.async_copy(idx_src(1), idx1, si1)
        pltpu.make_async_copy(idx_src(0), idx0, si0).wait()
        pltpu.async_copy(table_hbm.at[idx0], rows0, sg0)

        def group(g, carry):
            for b in range(2):
                c = g * 2 + b
                nxt = 1 - b

                pltpu.make_async_copy(
                    table_hbm.at[idx_b[b]], rows_b[b], sg[b]
                ).wait()

                def start_next():
                    pltpu.make_async_copy(
                        idx_src(c + 1), idx_b[nxt], si[nxt]
                    ).wait()
                    pltpu.make_async_copy(
                        rows_b[nxt], out_dst(c - 1), sw[nxt]
                    ).wait()
                    pltpu.async_copy(
                        table_hbm.at[idx_b[nxt]], rows_b[nxt], sg[nxt]
                    )

                if b == 0:
                    @pl.when(g > 0)
                    def _():
                        pltpu.make_async_copy(
                            rows_b[nxt], out_dst(c - 1), sw[nxt]
                        ).wait()

                    pltpu.make_async_copy(
                        idx_src(c + 1), idx_b[nxt], si[nxt]
                    ).wait()
                    pltpu.async_copy(
                        table_hbm.at[idx_b[nxt]], rows_b[nxt], sg[nxt]
                    )
                else:
                    @pl.when(g < n_groups - 1)
                    def _():
                        start_next()

                pltpu.async_copy(rows_b[b], out_dst(c), sw[b])

                @pl.when(g < n_groups - 1)
                def _():
                    pltpu.async_copy(idx_src(c + 2), idx_b[b], si[b])

            return carry

        lax.fori_loop(0, n_groups, group, ())

        for b in range(2):
            off = base + ((n_groups - 1) * 2 + b) * chunk
            pltpu.make_async_copy(
                rows_b[b], out_hbm.at[pl.ds(off, chunk)], sw[b]
            ).wait()

    return gather_kernel


def kernel(token_ids, table):
    B, S = token_ids.shape
    V, D = table.shape
    N = B * S
    info = plsc.get_sparse_core_info()
    NC, NS = info.num_cores, info.num_subcores
    ids_flat = token_ids.reshape(N).astype(jnp.int32)
    out = _make_gather(N, V, D, NC, NS, chunk=320)(ids_flat, table)
    return out.reshape(B, S, D)


# speedup vs baseline: 9.1928x; 1.0015x over previous
import functools

import jax
import jax.numpy as jnp
from jax import lax
from jax.experimental import pallas as pl
from jax.experimental.pallas import tpu as pltpu
from jax.experimental.pallas import tpu_sc as plsc


def _make_gather(N, V, D, NC, NS, chunk):
    NW = NC * NS
    n_per_w = N // NW
    n_chunks = n_per_w // chunk
    n_groups = n_chunks // 2
    mesh = plsc.VectorSubcoreMesh(core_axis_name="c", subcore_axis_name="s")

    @functools.partial(
        pl.kernel,
        mesh=mesh,
        out_type=jax.ShapeDtypeStruct((N, D), jnp.float32),
        scratch_types=[
            pltpu.VMEM((chunk,), jnp.int32),
            pltpu.VMEM((chunk,), jnp.int32),
            pltpu.VMEM((chunk, D), jnp.float32),
            pltpu.VMEM((chunk, D), jnp.float32),
            pltpu.SemaphoreType.DMA,
            pltpu.SemaphoreType.DMA,
            pltpu.SemaphoreType.DMA,
            pltpu.SemaphoreType.DMA,
            pltpu.SemaphoreType.DMA,
            pltpu.SemaphoreType.DMA,
        ],
    )
    def gather_kernel(idx_hbm, table_hbm, out_hbm, idx0, idx1, rows0, rows1,
                      sg0, sg1, sw0, sw1, si0, si1):
        wid = lax.axis_index("s") * NC + lax.axis_index("c")
        base = wid * n_per_w
        idx_b = (idx0, idx1)
        rows_b = (rows0, rows1)
        sg = (sg0, sg1)
        sw = (sw0, sw1)
        si = (si0, si1)

        def idx_src(c):
            return idx_hbm.at[pl.ds(base + c * chunk, chunk)]

        def out_dst(c):
            return out_hbm.at[pl.ds(base + c * chunk, chunk)]

        pltpu.async_copy(idx_src(0), idx0, si0)
        pltpu.async_copy(idx_src(1), idx1, si1)
        pltpu.make_async_copy(idx_src(0), idx0, si0).wait()
        pltpu.async_copy(table_hbm.at[idx0], rows0, sg0)

        def group(g, carry):
            for b in range(2):
                c = g * 2 + b
                nxt = 1 - b

                pltpu.make_async_copy(
                    table_hbm.at[idx_b[b]], rows_b[b], sg[b]
                ).wait()

                def start_next():
                    pltpu.make_async_copy(
                        idx_src(c + 1), idx_b[nxt], si[nxt]
                    ).wait()
                    pltpu.make_async_copy(
                        rows_b[nxt], out_dst(c - 1), sw[nxt]
                    ).wait()
                    pltpu.async_copy(
                        table_hbm.at[idx_b[nxt]], rows_b[nxt], sg[nxt]
                    )

                if b == 0:
                    @pl.when(g > 0)
                    def _():
                        pltpu.make_async_copy(
                            rows_b[nxt], out_dst(c - 1), sw[nxt]
                        ).wait()

                    pltpu.make_async_copy(
                        idx_src(c + 1), idx_b[nxt], si[nxt]
                    ).wait()
                    pltpu.async_copy(
                        table_hbm.at[idx_b[nxt]], rows_b[nxt], sg[nxt]
                    )
                else:
                    @pl.when(g < n_groups - 1)
                    def _():
                        start_next()

                pltpu.async_copy(rows_b[b], out_dst(c), sw[b])

                @pl.when(g < n_groups - 1)
                def _():
                    pltpu.async_copy(idx_src(c + 2), idx_b[b], si[b])

            return carry

        lax.fori_loop(0, n_groups, group, ())

        for b in range(2):
            off = base + ((n_groups - 1) * 2 + b) * chunk
            pltpu.make_async_copy(
                rows_b[b], out_hbm.at[pl.ds(off, chunk)], sw[b]
            ).wait()

    return gather_kernel


def kernel(token_ids, table):
    B, S = token_ids.shape
    V, D = table.shape
    N = B * S
    info = plsc.get_sparse_core_info()
    NC, NS = info.num_cores, info.num_subcores
    ids_flat = token_ids.reshape(N).astype(jnp.int32)
    out = _make_gather(N, V, D, NC, NS, chunk=400)(ids_flat, table)
    return out.reshape(B, S, D)
